# Initial kernel scaffold; baseline (speedup 1.0000x reference)
import jax
import jax.numpy as jnp
from jax.experimental import pallas as pl


def kernel(x, edge_index, W1, b1, W2, b2, W3, b3):
    raise NotImplementedError("write your pallas kernel here")


# speedup vs baseline: 15.6675x; 15.6675x over previous
import functools

import jax
import jax.numpy as jnp
from jax import lax
from jax.experimental import pallas as pl
from jax.experimental.pallas import tpu as pltpu
from jax.experimental.pallas import tpu_sc as plsc

N = 10000
E = 320000
NC, NS = 2, 16
NW = NC * NS
C = 128
NCH = 79
EPT = NCH * C
EPAD = NW * EPT
NPAD = 10240
RPT = NPAD // NS

_MESH = plsc.VectorSubcoreMesh(core_axis_name="c", subcore_axis_name="s")


def _make_prop(F):
    @functools.partial(
        pl.kernel,
        out_type=jax.ShapeDtypeStruct((2 * NPAD, F), jnp.float32),
        mesh=_MESH,
        compiler_params=pltpu.CompilerParams(use_tc_tiling_on_sc=False),
        scratch_types=[
            pltpu.VMEM((C,), jnp.int32),
            pltpu.VMEM((C,), jnp.int32),
            pltpu.VMEM((C, F), jnp.float32),
            pltpu.SemaphoreType.DMA,
            pltpu.VMEM_SHARED((NPAD, F), jnp.float32),
        ],
    )
    def prop(h_hbm, src_hbm, dst_hbm, out_hbm, idxs, idxd, rows, sem, acc):
        cid = lax.axis_index("c")
        sid = lax.axis_index("s")

        def zrow(i, carry):
            for c4 in range(F // 16):
                rows[i, pl.ds(c4 * 16, 16)] = jnp.zeros((16,), jnp.float32)
            return carry

        lax.fori_loop(0, C, zrow, 0)
        for r in range(RPT // C):
            pltpu.sync_copy(rows, acc.at[pl.ds(sid * RPT + r * C, C)])
        plsc.subcore_barrier()

        base0 = (sid * NC + cid) * EPT

        def body(i, carry):
            b = base0 + i * C
            pltpu.sync_copy(src_hbm.at[pl.ds(b, C)], idxs)
            pltpu.sync_copy(dst_hbm.at[pl.ds(b, C)], idxd)
            pltpu.async_copy(h_hbm.at[idxs], rows, sem).wait()
            pltpu.sync_copy(rows, acc.at[idxd], add=True)
            return carry

        lax.fori_loop(0, NCH, body, 0)
        plsc.subcore_barrier()
        pltpu.sync_copy(
            acc.at[pl.ds(sid * RPT, RPT)],
            out_hbm.at[pl.ds(cid * NPAD + sid * RPT, RPT)],
        )

    return prop


_prop64 = _make_prop(64)
_prop32 = _make_prop(32)
_prop16 = _make_prop(16)


@functools.partial(
    pl.kernel,
    out_type=jax.ShapeDtypeStruct((2 * NPAD,), jnp.float32),
    mesh=_MESH,
    scratch_types=[
        pltpu.VMEM((C,), jnp.int32),
        pltpu.VMEM((C,), jnp.float32),
        pltpu.VMEM_SHARED((NPAD,), jnp.float32),
    ],
)
def _deg_pass(dst_hbm, out_hbm, idxd, ones, acc):
    cid = lax.axis_index("c")
    sid = lax.axis_index("s")

    for c4 in range(C // 16):
        ones[pl.ds(c4 * 16, 16)] = jnp.zeros((16,), jnp.float32)
    for r in range(RPT // C):
        pltpu.sync_copy(ones, acc.at[pl.ds(sid * RPT + r * C, C)])
    plsc.subcore_barrier()
    for c4 in range(C // 16):
        ones[pl.ds(c4 * 16, 16)] = jnp.ones((16,), jnp.float32)

    base0 = (sid * NC + cid) * EPT

    def body(i, carry):
        b = base0 + i * C
        pltpu.sync_copy(dst_hbm.at[pl.ds(b, C)], idxd)
        pltpu.sync_copy(ones, acc.at[idxd], add=True)
        return carry

    lax.fori_loop(0, NCH, body, 0)
    plsc.subcore_barrier()
    pltpu.sync_copy(
        acc.at[pl.ds(sid * RPT, RPT)],
        out_hbm.at[pl.ds(cid * NPAD + sid * RPT, RPT)],
    )


_BM = 1000


def _mm1(x, W):
    M, K = x.shape
    F = W.shape[1]

    def body(xr, wr, o):
        o[...] = jnp.dot(xr[...], wr[...], preferred_element_type=jnp.float32)

    return pl.pallas_call(
        body,
        grid=(M // _BM,),
        in_specs=[
            pl.BlockSpec((_BM, K), lambda i: (i, 0)),
            pl.BlockSpec((K, F), lambda i: (0, 0)),
        ],
        out_specs=pl.BlockSpec((_BM, F), lambda i: (i, 0)),
        out_shape=jax.ShapeDtypeStruct((M, F), jnp.float32),
    )(x, W)


def _scale1(d0, d1, u1):
    M, F = u1.shape

    def body(d0r, d1r, ur, dr, hr):
        dinv = lax.rsqrt(d0r[...] + d1r[...] + 1.0)
        dr[...] = dinv
        hr[...] = dinv * ur[...]

    return pl.pallas_call(
        body,
        grid=(M // _BM,),
        in_specs=[
            pl.BlockSpec((_BM, 1), lambda i: (i, 0)),
            pl.BlockSpec((_BM, 1), lambda i: (i, 0)),
            pl.BlockSpec((_BM, F), lambda i: (i, 0)),
        ],
        out_specs=[
            pl.BlockSpec((_BM, 1), lambda i: (i, 0)),
            pl.BlockSpec((_BM, F), lambda i: (i, 0)),
        ],
        out_shape=[
            jax.ShapeDtypeStruct((M, 1), jnp.float32),
            jax.ShapeDtypeStruct((M, F), jnp.float32),
        ],
    )(d0, d1, u1)


def _layer(a0, a1, h, dinv, b, W):
    M, F = h.shape
    F2 = W.shape[1]

    def body(a0r, a1r, hr, dr, br, wr, o):
        t = dr[...] * (a0r[...] + a1r[...] + hr[...]) + br[...]
        t = jnp.maximum(t, 0.0)
        o[...] = dr[...] * jnp.dot(t, wr[...], preferred_element_type=jnp.float32)

    return pl.pallas_call(
        body,
        grid=(M // _BM,),
        in_specs=[
            pl.BlockSpec((_BM, F), lambda i: (i, 0)),
            pl.BlockSpec((_BM, F), lambda i: (i, 0)),
            pl.BlockSpec((_BM, F), lambda i: (i, 0)),
            pl.BlockSpec((_BM, 1), lambda i: (i, 0)),
            pl.BlockSpec((1, F), lambda i: (0, 0)),
            pl.BlockSpec((F, F2), lambda i: (0, 0)),
        ],
        out_specs=pl.BlockSpec((_BM, F2), lambda i: (i, 0)),
        out_shape=jax.ShapeDtypeStruct((M, F2), jnp.float32),
    )(a0, a1, h, dinv, b, W)


def _final(a0, a1, h, dinv, b):
    M, F = h.shape

    def body(a0r, a1r, hr, dr, br, o):
        t = dr[...] * (a0r[...] + a1r[...] + hr[...]) + br[...]
        o[...] = jax.nn.sigmoid(t)

    return pl.pallas_call(
        body,
        grid=(M // _BM,),
        in_specs=[
            pl.BlockSpec((_BM, F), lambda i: (i, 0)),
            pl.BlockSpec((_BM, F), lambda i: (i, 0)),
            pl.BlockSpec((_BM, F), lambda i: (i, 0)),
            pl.BlockSpec((_BM, 1), lambda i: (i, 0)),
            pl.BlockSpec((1, F), lambda i: (0, 0)),
        ],
        out_specs=pl.BlockSpec((_BM, F), lambda i: (i, 0)),
        out_shape=jax.ShapeDtypeStruct((M, F), jnp.float32),
    )(a0, a1, h, dinv, b)


def kernel(x, edge_index, W1, b1, W2, b2, W3, b3):
    ei = edge_index.astype(jnp.int32)
    pad = EPAD - E
    src = jnp.concatenate([ei[0], jnp.zeros((pad,), jnp.int32)])
    dst = jnp.concatenate([ei[1], jnp.full((pad,), N, jnp.int32)])

    degp = _deg_pass(dst)
    u1 = _mm1(x, W1)
    d0 = degp[:N].reshape(N, 1)
    d1 = degp[NPAD:NPAD + N].reshape(N, 1)
    dinv, h1 = _scale1(d0, d1, u1)

    accp = _prop64(h1, src, dst)
    h2 = _layer(accp[:N], accp[NPAD:NPAD + N], h1, dinv,
                b1.reshape(1, -1), W2)

    accp = _prop32(h2, src, dst)
    h3 = _layer(accp[:N], accp[NPAD:NPAD + N], h2, dinv,
                b2.reshape(1, -1), W3)

    accp = _prop16(h3, src, dst)
    return _final(accp[:N], accp[NPAD:NPAD + N], h3, dinv,
                  b3.reshape(1, -1))


# speedup vs baseline: 20.3451x; 1.2986x over previous
import functools

import jax
import jax.numpy as jnp
from jax import lax
from jax.experimental import pallas as pl
from jax.experimental.pallas import tpu as pltpu
from jax.experimental.pallas import tpu_sc as plsc

N = 10000
E = 320000
NC, NS = 2, 16
NW = NC * NS
C = 128
NCH = 80
EPT = NCH * C
EPAD = NW * EPT
NPAD = 10240
RPT = NPAD // NS
NB = 4
GRPS = NCH // NB

_MESH = plsc.VectorSubcoreMesh(core_axis_name="c", subcore_axis_name="s")
_SC_PARAMS = pltpu.CompilerParams(use_tc_tiling_on_sc=False)


def _make_prop(F):
    @functools.partial(
        pl.kernel,
        out_type=jax.ShapeDtypeStruct((2 * NPAD, F), jnp.float32),
        mesh=_MESH,
        compiler_params=_SC_PARAMS,
        scratch_types=[
            pltpu.VMEM((NCH, C), jnp.int32),
            pltpu.VMEM((NCH, C), jnp.int32),
            pltpu.VMEM((2, NB, C, F), jnp.float32),
            pltpu.SemaphoreType.DMA,
            pltpu.SemaphoreType.DMA,
            pltpu.SemaphoreType.DMA,
            pltpu.SemaphoreType.DMA,
            pltpu.VMEM_SHARED((NPAD, F), jnp.float32),
        ],
    )
    def prop(h_hbm, src_hbm, dst_hbm, out_hbm, srcv, dstv, rows, g0, g1,
             s0, s1, acc):
        cid = lax.axis_index("c")
        sid = lax.axis_index("s")
        wid = sid * NC + cid
        gsem = (g0, g1)
        ssem = (s0, s1)

        pltpu.sync_copy(src_hbm.at[wid], srcv)
        pltpu.sync_copy(dst_hbm.at[wid], dstv)

        def zrow(i, carry):
            for c4 in range(F // 16):
                rows[0, 0, i, pl.ds(c4 * 16, 16)] = jnp.zeros(
                    (16,), jnp.float32)
            return carry

        lax.fori_loop(0, C, zrow, 0)
        for r in range(RPT // C):
            pltpu.sync_copy(rows.at[0, 0],
                            acc.at[pl.ds(sid * RPT + r * C, C)])
        plsc.subcore_barrier()

        def fire_g(slot, grp):
            for b in range(NB):
                i = grp * NB + b
                pltpu.async_copy(h_hbm.at[srcv.at[i]], rows.at[slot, b],
                                 gsem[slot])

        def wait_g(slot, grp):
            for b in range(NB):
                i = grp * NB + b
                pltpu.make_async_copy(h_hbm.at[srcv.at[i]],
                                      rows.at[slot, b], gsem[slot]).wait()

        def run_s(slot, grp):
            ds = []
            for b in range(NB):
                i = grp * NB + b
                ds.append(pltpu.async_copy(rows.at[slot, b],
                                           acc.at[dstv.at[i]], ssem[slot],
                                           add=True))
            for d in ds:
                d.wait()

        fire_g(0, 0)
        fire_g(1, 1)

        def outer(j2, carry):
            j = j2 * 2
            wait_g(0, j)
            run_s(0, j)
            fire_g(0, lax.rem(j + 2, GRPS))
            wait_g(1, j + 1)
            run_s(1, j + 1)
            fire_g(1, lax.rem(j + 3, GRPS))
            return carry

        lax.fori_loop(0, GRPS // 2, outer, 0)
        wait_g(0, 0)
        wait_g(1, 1)
        plsc.subcore_barrier()
        pltpu.sync_copy(
            acc.at[pl.ds(sid * RPT, RPT)],
            out_hbm.at[pl.ds(cid * NPAD + sid * RPT, RPT)],
        )

    return prop


_prop64 = _make_prop(64)
_prop32 = _make_prop(32)
_prop16 = _make_prop(16)


@functools.partial(
    pl.kernel,
    out_type=jax.ShapeDtypeStruct((2 * NPAD,), jnp.float32),
    mesh=_MESH,
    compiler_params=_SC_PARAMS,
    scratch_types=[
        pltpu.VMEM((NCH, C), jnp.int32),
        pltpu.VMEM((C,), jnp.float32),
        pltpu.SemaphoreType.DMA,
        pltpu.VMEM_SHARED((NPAD,), jnp.float32),
    ],
)
def _deg_pass(dst_hbm, out_hbm, dstv, ones, sem, acc):
    cid = lax.axis_index("c")
    sid = lax.axis_index("s")
    wid = sid * NC + cid

    pltpu.sync_copy(dst_hbm.at[wid], dstv)
    for c4 in range(C // 16):
        ones[pl.ds(c4 * 16, 16)] = jnp.zeros((16,), jnp.float32)
    for r in range(RPT // C):
        pltpu.sync_copy(ones, acc.at[pl.ds(sid * RPT + r * C, C)])
    plsc.subcore_barrier()
    for c4 in range(C // 16):
        ones[pl.ds(c4 * 16, 16)] = jnp.ones((16,), jnp.float32)

    def body(j, carry):
        ds = []
        for b in range(8):
            i = j * 8 + b
            ds.append(pltpu.async_copy(ones, acc.at[dstv.at[i]], sem,
                                       add=True))
        for d in ds:
            d.wait()
        return carry

    lax.fori_loop(0, NCH // 8, body, 0)
    plsc.subcore_barrier()
    pltpu.sync_copy(
        acc.at[pl.ds(sid * RPT, RPT)],
        out_hbm.at[pl.ds(cid * NPAD + sid * RPT, RPT)],
    )


_BM = 1000


def _mm1(x, W):
    M, K = x.shape
    F = W.shape[1]

    def body(xr, wr, o):
        o[...] = jnp.dot(xr[...], wr[...], preferred_element_type=jnp.float32)

    return pl.pallas_call(
        body,
        grid=(M // _BM,),
        in_specs=[
            pl.BlockSpec((_BM, K), lambda i: (i, 0)),
            pl.BlockSpec((K, F), lambda i: (0, 0)),
        ],
        out_specs=pl.BlockSpec((_BM, F), lambda i: (i, 0)),
        out_shape=jax.ShapeDtypeStruct((M, F), jnp.float32),
    )(x, W)


def _scale1(d0, d1, u1):
    M, F = u1.shape

    def body(d0r, d1r, ur, dr, hr):
        dinv = lax.rsqrt(d0r[...] + d1r[...] + 1.0)
        dr[...] = dinv
        hr[...] = dinv * ur[...]

    return pl.pallas_call(
        body,
        grid=(M // _BM,),
        in_specs=[
            pl.BlockSpec((_BM, 1), lambda i: (i, 0)),
            pl.BlockSpec((_BM, 1), lambda i: (i, 0)),
            pl.BlockSpec((_BM, F), lambda i: (i, 0)),
        ],
        out_specs=[
            pl.BlockSpec((_BM, 1), lambda i: (i, 0)),
            pl.BlockSpec((_BM, F), lambda i: (i, 0)),
        ],
        out_shape=[
            jax.ShapeDtypeStruct((M, 1), jnp.float32),
            jax.ShapeDtypeStruct((M, F), jnp.float32),
        ],
    )(d0, d1, u1)


def _layer(a0, a1, h, dinv, b, W):
    M, F = h.shape
    F2 = W.shape[1]

    def body(a0r, a1r, hr, dr, br, wr, o):
        t = dr[...] * (a0r[...] + a1r[...] + hr[...]) + br[...]
        t = jnp.maximum(t, 0.0)
        o[...] = dr[...] * jnp.dot(t, wr[...],
                                   preferred_element_type=jnp.float32)

    return pl.pallas_call(
        body,
        grid=(M // _BM,),
        in_specs=[
            pl.BlockSpec((_BM, F), lambda i: (i, 0)),
            pl.BlockSpec((_BM, F), lambda i: (i, 0)),
            pl.BlockSpec((_BM, F), lambda i: (i, 0)),
            pl.BlockSpec((_BM, 1), lambda i: (i, 0)),
            pl.BlockSpec((1, F), lambda i: (0, 0)),
            pl.BlockSpec((F, F2), lambda i: (0, 0)),
        ],
        out_specs=pl.BlockSpec((_BM, F2), lambda i: (i, 0)),
        out_shape=jax.ShapeDtypeStruct((M, F2), jnp.float32),
    )(a0, a1, h, dinv, b, W)


def _final(a0, a1, h, dinv, b):
    M, F = h.shape

    def body(a0r, a1r, hr, dr, br, o):
        t = dr[...] * (a0r[...] + a1r[...] + hr[...]) + br[...]
        o[...] = jax.nn.sigmoid(t)

    return pl.pallas_call(
        body,
        grid=(M // _BM,),
        in_specs=[
            pl.BlockSpec((_BM, F), lambda i: (i, 0)),
            pl.BlockSpec((_BM, F), lambda i: (i, 0)),
            pl.BlockSpec((_BM, F), lambda i: (i, 0)),
            pl.BlockSpec((_BM, 1), lambda i: (i, 0)),
            pl.BlockSpec((1, F), lambda i: (0, 0)),
        ],
        out_specs=pl.BlockSpec((_BM, F), lambda i: (i, 0)),
        out_shape=jax.ShapeDtypeStruct((M, F), jnp.float32),
    )(a0, a1, h, dinv, b)


def kernel(x, edge_index, W1, b1, W2, b2, W3, b3):
    ei = edge_index.astype(jnp.int32)
    pad = EPAD - E
    src = jnp.concatenate([ei[0], jnp.zeros((pad,), jnp.int32)])
    dst = jnp.concatenate([ei[1], jnp.full((pad,), N, jnp.int32)])
    src = src.reshape(NW, NCH, C)
    dst = dst.reshape(NW, NCH, C)

    degp = _deg_pass(dst)
    u1 = _mm1(x, W1)
    d0 = degp[:N].reshape(N, 1)
    d1 = degp[NPAD:NPAD + N].reshape(N, 1)
    dinv, h1 = _scale1(d0, d1, u1)

    accp = _prop64(h1, src, dst)
    h2 = _layer(accp[:N], accp[NPAD:NPAD + N], h1, dinv,
                b1.reshape(1, -1), W2)

    accp = _prop32(h2, src, dst)
    h3 = _layer(accp[:N], accp[NPAD:NPAD + N], h2, dinv,
                b2.reshape(1, -1), W3)

    accp = _prop16(h3, src, dst)
    return _final(accp[:N], accp[NPAD:NPAD + N], h3, dinv,
                  b3.reshape(1, -1))


# speedup vs baseline: 20.5405x; 1.0096x over previous
import functools

import jax
import jax.numpy as jnp
from jax import lax
from jax.experimental import pallas as pl
from jax.experimental.pallas import tpu as pltpu
from jax.experimental.pallas import tpu_sc as plsc

N = 10000
E = 320000
NC, NS = 2, 16
NW = NC * NS
C = 128
NCH = 80
EPT = NCH * C
EPAD = NW * EPT
NPAD = 10240
RPT = NPAD // NS
NB = 4
GRPS = NCH // NB

_MESH = plsc.VectorSubcoreMesh(core_axis_name="c", subcore_axis_name="s")
_SC_PARAMS = pltpu.CompilerParams(use_tc_tiling_on_sc=False)


def _make_prop(F):
    @functools.partial(
        pl.kernel,
        out_type=jax.ShapeDtypeStruct((2 * NPAD, F), jnp.float32),
        mesh=_MESH,
        compiler_params=_SC_PARAMS,
        scratch_types=[
            pltpu.VMEM((NCH, C), jnp.int32),
            pltpu.VMEM((NCH, C), jnp.int32),
            pltpu.VMEM((2, NB, C, F), jnp.float32),
            pltpu.SemaphoreType.DMA,
            pltpu.SemaphoreType.DMA,
            pltpu.SemaphoreType.DMA,
            pltpu.SemaphoreType.DMA,
            pltpu.VMEM_SHARED((NPAD, F), jnp.float32),
        ],
    )
    def prop(h_hbm, src_hbm, dst_hbm, out_hbm, srcv, dstv, rows, g0, g1,
             s0, s1, acc):
        cid = lax.axis_index("c")
        sid = lax.axis_index("s")
        wid = sid * NC + cid
        gsem = (g0, g1)
        ssem = (s0, s1)

        pltpu.sync_copy(src_hbm.at[wid], srcv)
        pltpu.sync_copy(dst_hbm.at[wid], dstv)

        def zrow(i, carry):
            for c4 in range(F // 16):
                rows[0, 0, i, pl.ds(c4 * 16, 16)] = jnp.zeros(
                    (16,), jnp.float32)
            return carry

        lax.fori_loop(0, C, zrow, 0)
        for r in range(RPT // C):
            pltpu.sync_copy(rows.at[0, 0],
                            acc.at[pl.ds(sid * RPT + r * C, C)])
        plsc.subcore_barrier()

        def fire_g(slot, grp):
            for b in range(NB):
                i = grp * NB + b
                pltpu.async_copy(h_hbm.at[srcv.at[i]], rows.at[slot, b],
                                 gsem[slot])

        def wait_g(slot, grp):
            for b in range(NB):
                i = grp * NB + b
                pltpu.make_async_copy(h_hbm.at[srcv.at[i]],
                                      rows.at[slot, b], gsem[slot]).wait()

        def run_s(slot, grp):
            ds = []
            for b in range(NB):
                i = grp * NB + b
                ds.append(pltpu.async_copy(rows.at[slot, b],
                                           acc.at[dstv.at[i]], ssem[slot],
                                           add=True))
            for d in ds:
                d.wait()

        fire_g(0, 0)
        fire_g(1, 1)

        def outer(j2, carry):
            j = j2 * 2
            wait_g(0, j)
            run_s(0, j)
            fire_g(0, lax.rem(j + 2, GRPS))
            wait_g(1, j + 1)
            run_s(1, j + 1)
            fire_g(1, lax.rem(j + 3, GRPS))
            return carry

        lax.fori_loop(0, GRPS // 2, outer, 0)
        wait_g(0, 0)
        wait_g(1, 1)
        plsc.subcore_barrier()
        pltpu.sync_copy(
            acc.at[pl.ds(sid * RPT, RPT)],
            out_hbm.at[pl.ds(cid * NPAD + sid * RPT, RPT)],
        )

    return prop


_prop64 = _make_prop(64)
_prop32 = _make_prop(32)
_prop16 = _make_prop(16)


@functools.partial(
    pl.kernel,
    out_type=jax.ShapeDtypeStruct((2 * NPAD,), jnp.float32),
    mesh=_MESH,
    compiler_params=_SC_PARAMS,
    scratch_types=[
        pltpu.VMEM((NCH, C), jnp.int32),
        pltpu.VMEM((C,), jnp.float32),
        pltpu.SemaphoreType.DMA,
        pltpu.VMEM_SHARED((NPAD,), jnp.float32),
    ],
)
def _deg_pass(dst_hbm, out_hbm, dstv, ones, sem, acc):
    cid = lax.axis_index("c")
    sid = lax.axis_index("s")
    wid = sid * NC + cid

    pltpu.sync_copy(dst_hbm.at[wid], dstv)
    for c4 in range(C // 16):
        ones[pl.ds(c4 * 16, 16)] = jnp.zeros((16,), jnp.float32)
    for r in range(RPT // C):
        pltpu.sync_copy(ones, acc.at[pl.ds(sid * RPT + r * C, C)])
    plsc.subcore_barrier()
    for c4 in range(C // 16):
        ones[pl.ds(c4 * 16, 16)] = jnp.ones((16,), jnp.float32)

    def body(j, carry):
        ds = []
        for b in range(8):
            i = j * 8 + b
            ds.append(pltpu.async_copy(ones, acc.at[dstv.at[i]], sem,
                                       add=True))
        for d in ds:
            d.wait()
        return carry

    lax.fori_loop(0, NCH // 8, body, 0)
    plsc.subcore_barrier()
    pltpu.sync_copy(
        acc.at[pl.ds(sid * RPT, RPT)],
        out_hbm.at[pl.ds(cid * NPAD + sid * RPT, RPT)],
    )


_BM = 1000


def _mm1(x, W):
    M, K = x.shape
    F = W.shape[1]

    def body(xr, wr, o):
        o[...] = jnp.dot(xr[...], wr[...], preferred_element_type=jnp.float32)

    return pl.pallas_call(
        body,
        grid=(M // _BM,),
        in_specs=[
            pl.BlockSpec((_BM, K), lambda i: (i, 0)),
            pl.BlockSpec((K, F), lambda i: (0, 0)),
        ],
        out_specs=pl.BlockSpec((_BM, F), lambda i: (i, 0)),
        out_shape=jax.ShapeDtypeStruct((M, F), jnp.float32),
    )(x, W)


def _scale1(d0, d1, u1):
    M, F = u1.shape

    def body(d0r, d1r, ur, dr, hr):
        dinv = lax.rsqrt(d0r[...] + d1r[...] + 1.0)
        dr[...] = dinv
        hr[...] = dinv * ur[...]

    return pl.pallas_call(
        body,
        grid=(M // _BM,),
        in_specs=[
            pl.BlockSpec((_BM, 1), lambda i: (i, 0)),
            pl.BlockSpec((_BM, 1), lambda i: (i, 0)),
            pl.BlockSpec((_BM, F), lambda i: (i, 0)),
        ],
        out_specs=[
            pl.BlockSpec((_BM, 1), lambda i: (i, 0)),
            pl.BlockSpec((_BM, F), lambda i: (i, 0)),
        ],
        out_shape=[
            jax.ShapeDtypeStruct((M, 1), jnp.float32),
            jax.ShapeDtypeStruct((M, F), jnp.float32),
        ],
    )(d0, d1, u1)


def _layer(a0, a1, h, dinv, b, W):
    M, F = h.shape
    F2 = W.shape[1]

    def body(a0r, a1r, hr, dr, br, wr, o):
        t = dr[...] * (a0r[...] + a1r[...] + hr[...]) + br[...]
        t = jnp.maximum(t, 0.0)
        o[...] = dr[...] * jnp.dot(t, wr[...],
                                   preferred_element_type=jnp.float32)

    return pl.pallas_call(
        body,
        grid=(M // _BM,),
        in_specs=[
            pl.BlockSpec((_BM, F), lambda i: (i, 0)),
            pl.BlockSpec((_BM, F), lambda i: (i, 0)),
            pl.BlockSpec((_BM, F), lambda i: (i, 0)),
            pl.BlockSpec((_BM, 1), lambda i: (i, 0)),
            pl.BlockSpec((1, F), lambda i: (0, 0)),
            pl.BlockSpec((F, F2), lambda i: (0, 0)),
        ],
        out_specs=pl.BlockSpec((_BM, F2), lambda i: (i, 0)),
        out_shape=jax.ShapeDtypeStruct((M, F2), jnp.float32),
    )(a0, a1, h, dinv, b, W)


def _final(a0, a1, h, dinv, b):
    M, F = h.shape

    def body(a0r, a1r, hr, dr, br, o):
        t = dr[...] * (a0r[...] + a1r[...] + hr[...]) + br[...]
        o[...] = jax.nn.sigmoid(t)

    return pl.pallas_call(
        body,
        grid=(M // _BM,),
        in_specs=[
            pl.BlockSpec((_BM, F), lambda i: (i, 0)),
            pl.BlockSpec((_BM, F), lambda i: (i, 0)),
            pl.BlockSpec((_BM, F), lambda i: (i, 0)),
            pl.BlockSpec((_BM, 1), lambda i: (i, 0)),
            pl.BlockSpec((1, F), lambda i: (0, 0)),
        ],
        out_specs=pl.BlockSpec((_BM, F), lambda i: (i, 0)),
        out_shape=jax.ShapeDtypeStruct((M, F), jnp.float32),
    )(a0, a1, h, dinv, b)


def kernel(x, edge_index, W1, b1, W2, b2, W3, b3):
    ei = edge_index.astype(jnp.int32)
    pad = EPAD - E
    src = jnp.concatenate([ei[0], jnp.zeros((pad,), jnp.int32)])
    dst = jnp.concatenate(
        [ei[1], N + (jnp.arange(pad, dtype=jnp.int32) % (NPAD - N))])
    src = src.reshape(NW, NCH, C)
    dst = dst.reshape(NW, NCH, C)

    degp = _deg_pass(dst)
    u1 = _mm1(x, W1)
    d0 = degp[:N].reshape(N, 1)
    d1 = degp[NPAD:NPAD + N].reshape(N, 1)
    dinv, h1 = _scale1(d0, d1, u1)

    accp = _prop64(h1, src, dst)
    h2 = _layer(accp[:N], accp[NPAD:NPAD + N], h1, dinv,
                b1.reshape(1, -1), W2)

    accp = _prop32(h2, src, dst)
    h3 = _layer(accp[:N], accp[NPAD:NPAD + N], h2, dinv,
                b2.reshape(1, -1), W3)

    accp = _prop16(h3, src, dst)
    return _final(accp[:N], accp[NPAD:NPAD + N], h3, dinv,
                  b3.reshape(1, -1))


# speedup vs baseline: 35.3906x; 1.7230x over previous
import functools

import jax
import jax.numpy as jnp
from jax import lax
from jax.experimental import pallas as pl
from jax.experimental.pallas import tpu as pltpu
from jax.experimental.pallas import tpu_sc as plsc

N = 10000
E = 320000
NC, NS = 2, 16
NW = NC * NS
C = 128
NCH = 80
EPT = NCH * C
EPAD = NW * EPT
NPAD = 10240
RPT = NPAD // NS
HRPT = N // NS
NB = 4
GRPS = NCH // NB

_MESH = plsc.VectorSubcoreMesh(core_axis_name="c", subcore_axis_name="s")
_SC_PARAMS = pltpu.CompilerParams(use_tc_tiling_on_sc=False)


def _make_prop(FB, ncb):
    @functools.partial(
        pl.kernel,
        out_type=jax.ShapeDtypeStruct((ncb * 2 * NPAD, FB), jnp.float32),
        mesh=_MESH,
        compiler_params=_SC_PARAMS,
        scratch_types=[
            pltpu.VMEM((NCH, C), jnp.int32),
            pltpu.VMEM((NCH, C), jnp.int32),
            pltpu.VMEM((2, NB, C, FB), jnp.float32),
            pltpu.VMEM((C, FB), jnp.float32),
            pltpu.SemaphoreType.DMA,
            pltpu.SemaphoreType.DMA,
            pltpu.SemaphoreType.DMA,
            pltpu.SemaphoreType.DMA,
            pltpu.SemaphoreType.DMA,
            pltpu.VMEM_SHARED((NPAD, FB), jnp.float32),
            pltpu.VMEM_SHARED((N, FB), jnp.float32),
        ],
    )
    def prop(*refs):
        h_hbm = refs[:ncb]
        src_hbm, dst_hbm, out_hbm = refs[ncb:ncb + 3]
        (srcv, dstv, rows, zbuf, g0, g1, s0, s1, hsem, acc, hsp) = \
            refs[ncb + 3:]
        cid = lax.axis_index("c")
        sid = lax.axis_index("s")
        wid = sid * NC + cid
        gsem = (g0, g1)
        ssem = (s0, s1)

        pltpu.sync_copy(src_hbm.at[wid], srcv)
        pltpu.sync_copy(dst_hbm.at[wid], dstv)

        def zrow(i, carry):
            for c4 in range(FB // 16):
                zbuf[i, pl.ds(c4 * 16, 16)] = jnp.zeros((16,), jnp.float32)
            return carry

        lax.fori_loop(0, C, zrow, 0)

        def fire_g(slot, grp):
            for b in range(NB):
                i = grp * NB + b
                pltpu.async_copy(hsp.at[srcv.at[i]], rows.at[slot, b],
                                 gsem[slot])

        def wait_g(slot, grp):
            for b in range(NB):
                i = grp * NB + b
                pltpu.make_async_copy(hsp.at[srcv.at[i]],
                                      rows.at[slot, b], gsem[slot]).wait()

        def run_s(slot, grp):
            ds = []
            for b in range(NB):
                i = grp * NB + b
                ds.append(pltpu.async_copy(rows.at[slot, b],
                                           acc.at[dstv.at[i]], ssem[slot],
                                           add=True))
            for d in ds:
                d.wait()

        for cb in range(ncb):
            stg = pltpu.async_copy(
                h_hbm[cb].at[pl.ds(sid * HRPT, HRPT)],
                hsp.at[pl.ds(sid * HRPT, HRPT)], hsem)
            for r in range(RPT // C):
                pltpu.sync_copy(zbuf, acc.at[pl.ds(sid * RPT + r * C, C)])
            stg.wait()
            plsc.subcore_barrier()

            fire_g(0, 0)
            fire_g(1, 1)

            def outer(j2, carry):
                j = j2 * 2
                wait_g(0, j)
                run_s(0, j)
                fire_g(0, lax.rem(j + 2, GRPS))
                wait_g(1, j + 1)
                run_s(1, j + 1)
                fire_g(1, lax.rem(j + 3, GRPS))
                return carry

            lax.fori_loop(0, GRPS // 2, outer, 0)
            wait_g(0, 0)
            wait_g(1, 1)
            plsc.subcore_barrier()
            pltpu.sync_copy(
                acc.at[pl.ds(sid * RPT, RPT)],
                out_hbm.at[pl.ds((cb * NC + cid) * NPAD + sid * RPT, RPT)],
            )
            if cb + 1 < ncb:
                plsc.subcore_barrier()

    return prop


_prop64 = _make_prop(32, 2)
_prop32 = _make_prop(32, 1)
_prop16 = _make_prop(16, 1)


@functools.partial(
    pl.kernel,
    out_type=jax.ShapeDtypeStruct((2 * NPAD,), jnp.float32),
    mesh=_MESH,
    compiler_params=_SC_PARAMS,
    scratch_types=[
        pltpu.VMEM((NCH, C), jnp.int32),
        pltpu.VMEM((C,), jnp.float32),
        pltpu.SemaphoreType.DMA,
        pltpu.VMEM_SHARED((NPAD,), jnp.float32),
    ],
)
def _deg_pass(dst_hbm, out_hbm, dstv, ones, sem, acc):
    cid = lax.axis_index("c")
    sid = lax.axis_index("s")
    wid = sid * NC + cid

    pltpu.sync_copy(dst_hbm.at[wid], dstv)
    for c4 in range(C // 16):
        ones[pl.ds(c4 * 16, 16)] = jnp.zeros((16,), jnp.float32)
    for r in range(RPT // C):
        pltpu.sync_copy(ones, acc.at[pl.ds(sid * RPT + r * C, C)])
    plsc.subcore_barrier()
    for c4 in range(C // 16):
        ones[pl.ds(c4 * 16, 16)] = jnp.ones((16,), jnp.float32)

    def body(j, carry):
        ds = []
        for b in range(8):
            i = j * 8 + b
            ds.append(pltpu.async_copy(ones, acc.at[dstv.at[i]], sem,
                                       add=True))
        for d in ds:
            d.wait()
        return carry

    lax.fori_loop(0, NCH // 8, body, 0)
    plsc.subcore_barrier()
    pltpu.sync_copy(
        acc.at[pl.ds(sid * RPT, RPT)],
        out_hbm.at[pl.ds(cid * NPAD + sid * RPT, RPT)],
    )


_BM = 1000


def _mm1(x, W):
    M, K = x.shape
    F = W.shape[1]

    def body(xr, wr, o):
        o[...] = jnp.dot(xr[...], wr[...], preferred_element_type=jnp.float32)

    return pl.pallas_call(
        body,
        grid=(M // _BM,),
        in_specs=[
            pl.BlockSpec((_BM, K), lambda i: (i, 0)),
            pl.BlockSpec((K, F), lambda i: (0, 0)),
        ],
        out_specs=pl.BlockSpec((_BM, F), lambda i: (i, 0)),
        out_shape=jax.ShapeDtypeStruct((M, F), jnp.float32),
    )(x, W)


def _scale1(d0, d1, u1):
    M, F = u1.shape

    def body(d0r, d1r, ur, dr, hl, hr):
        dinv = lax.rsqrt(d0r[...] + d1r[...] + 1.0)
        dr[...] = dinv
        h = dinv * ur[...]
        hl[...] = h[:, :F // 2]
        hr[...] = h[:, F // 2:]

    return pl.pallas_call(
        body,
        grid=(M // _BM,),
        in_specs=[
            pl.BlockSpec((_BM, 1), lambda i: (i, 0)),
            pl.BlockSpec((_BM, 1), lambda i: (i, 0)),
            pl.BlockSpec((_BM, F), lambda i: (i, 0)),
        ],
        out_specs=[
            pl.BlockSpec((_BM, 1), lambda i: (i, 0)),
            pl.BlockSpec((_BM, F // 2), lambda i: (i, 0)),
            pl.BlockSpec((_BM, F // 2), lambda i: (i, 0)),
        ],
        out_shape=[
            jax.ShapeDtypeStruct((M, 1), jnp.float32),
            jax.ShapeDtypeStruct((M, F // 2), jnp.float32),
            jax.ShapeDtypeStruct((M, F // 2), jnp.float32),
        ],
    )(d0, d1, u1)


def _layer1(al0, al1, ar0, ar1, hl, hr, dinv, b, W):
    M, FB = hl.shape
    F2 = W.shape[1]

    def body(al0r, al1r, ar0r, ar1r, hlr, hrr, dr, br, wr, o):
        tl = al0r[...] + al1r[...] + hlr[...]
        tr = ar0r[...] + ar1r[...] + hrr[...]
        t = dr[...] * jnp.concatenate([tl, tr], axis=1) + br[...]
        t = jnp.maximum(t, 0.0)
        o[...] = dr[...] * jnp.dot(t, wr[...],
                                   preferred_element_type=jnp.float32)

    bs = lambda s: pl.BlockSpec(s, lambda i: (i, 0))
    return pl.pallas_call(
        body,
        grid=(M // _BM,),
        in_specs=[
            bs((_BM, FB)), bs((_BM, FB)), bs((_BM, FB)), bs((_BM, FB)),
            bs((_BM, FB)), bs((_BM, FB)), bs((_BM, 1)),
            pl.BlockSpec((1, 2 * FB), lambda i: (0, 0)),
            pl.BlockSpec((2 * FB, F2), lambda i: (0, 0)),
        ],
        out_specs=pl.BlockSpec((_BM, F2), lambda i: (i, 0)),
        out_shape=jax.ShapeDtypeStruct((M, F2), jnp.float32),
    )(al0, al1, ar0, ar1, hl, hr, dinv, b, W)


def _layer(a0, a1, h, dinv, b, W):
    M, F = h.shape
    F2 = W.shape[1]

    def body(a0r, a1r, hr, dr, br, wr, o):
        t = dr[...] * (a0r[...] + a1r[...] + hr[...]) + br[...]
        t = jnp.maximum(t, 0.0)
        o[...] = dr[...] * jnp.dot(t, wr[...],
                                   preferred_element_type=jnp.float32)

    return pl.pallas_call(
        body,
        grid=(M // _BM,),
        in_specs=[
            pl.BlockSpec((_BM, F), lambda i: (i, 0)),
            pl.BlockSpec((_BM, F), lambda i: (i, 0)),
            pl.BlockSpec((_BM, F), lambda i: (i, 0)),
            pl.BlockSpec((_BM, 1), lambda i: (i, 0)),
            pl.BlockSpec((1, F), lambda i: (0, 0)),
            pl.BlockSpec((F, F2), lambda i: (0, 0)),
        ],
        out_specs=pl.BlockSpec((_BM, F2), lambda i: (i, 0)),
        out_shape=jax.ShapeDtypeStruct((M, F2), jnp.float32),
    )(a0, a1, h, dinv, b, W)


def _final(a0, a1, h, dinv, b):
    M, F = h.shape

    def body(a0r, a1r, hr, dr, br, o):
        t = dr[...] * (a0r[...] + a1r[...] + hr[...]) + br[...]
        o[...] = jax.nn.sigmoid(t)

    return pl.pallas_call(
        body,
        grid=(M // _BM,),
        in_specs=[
            pl.BlockSpec((_BM, F), lambda i: (i, 0)),
            pl.BlockSpec((_BM, F), lambda i: (i, 0)),
            pl.BlockSpec((_BM, F), lambda i: (i, 0)),
            pl.BlockSpec((_BM, 1), lambda i: (i, 0)),
            pl.BlockSpec((1, F), lambda i: (0, 0)),
        ],
        out_specs=pl.BlockSpec((_BM, F), lambda i: (i, 0)),
        out_shape=jax.ShapeDtypeStruct((M, F), jnp.float32),
    )(a0, a1, h, dinv, b)


def kernel(x, edge_index, W1, b1, W2, b2, W3, b3):
    ei = edge_index.astype(jnp.int32)
    pad = EPAD - E
    src = jnp.concatenate([ei[0], jnp.zeros((pad,), jnp.int32)])
    dst = jnp.concatenate(
        [ei[1], N + (jnp.arange(pad, dtype=jnp.int32) % (NPAD - N))])
    src = src.reshape(NW, NCH, C)
    dst = dst.reshape(NW, NCH, C)

    degp = _deg_pass(dst)
    u1 = _mm1(x, W1)
    d0 = degp[:N].reshape(N, 1)
    d1 = degp[NPAD:NPAD + N].reshape(N, 1)
    dinv, h1l, h1r = _scale1(d0, d1, u1)

    a = _prop64(h1l, h1r, src, dst)
    h2 = _layer1(a[:N], a[NPAD:NPAD + N],
                 a[2 * NPAD:2 * NPAD + N], a[3 * NPAD:3 * NPAD + N],
                 h1l, h1r, dinv, b1.reshape(1, -1), W2)

    a = _prop32(h2, src, dst)
    h3 = _layer(a[:N], a[NPAD:NPAD + N], h2, dinv,
                b2.reshape(1, -1), W3)

    a = _prop16(h3, src, dst)
    return _final(a[:N], a[NPAD:NPAD + N], h3, dinv,
                  b3.reshape(1, -1))


# speedup vs baseline: 39.1652x; 1.1067x over previous
import functools

import jax
import jax.numpy as jnp
from jax import lax
from jax.experimental import pallas as pl
from jax.experimental.pallas import tpu as pltpu
from jax.experimental.pallas import tpu_sc as plsc

N = 10000
E = 320000
NC, NS = 2, 16
NW = NC * NS
C = 128
NCH = 80
EPT = NCH * C
EPAD = NW * EPT
NPAD = 10240
RPT = NPAD // NS
NB = 4
GRPS = NCH // NB
_BM = 1024

_MESH = plsc.VectorSubcoreMesh(core_axis_name="c", subcore_axis_name="s")
_SC_PARAMS = pltpu.CompilerParams(use_tc_tiling_on_sc=False)


def _make_prop(FB, ncb):
    @functools.partial(
        pl.kernel,
        out_type=jax.ShapeDtypeStruct((ncb * 2 * NPAD, FB), jnp.float32),
        mesh=_MESH,
        compiler_params=_SC_PARAMS,
        scratch_types=[
            pltpu.VMEM((NCH, C), jnp.int32),
            pltpu.VMEM((NCH, C), jnp.int32),
            pltpu.VMEM((2, NB, C, FB), jnp.float32),
            pltpu.VMEM((C, FB), jnp.float32),
            pltpu.SemaphoreType.DMA,
            pltpu.SemaphoreType.DMA,
            pltpu.SemaphoreType.DMA,
            pltpu.SemaphoreType.DMA,
            pltpu.SemaphoreType.DMA,
            pltpu.VMEM_SHARED((NPAD, FB), jnp.float32),
            pltpu.VMEM_SHARED((NPAD, FB), jnp.float32),
        ],
    )
    def prop(*refs):
        h_hbm = refs[:ncb]
        src_hbm, dst_hbm, out_hbm = refs[ncb:ncb + 3]
        (srcv, dstv, rows, zbuf, g0, g1, s0, s1, hsem, acc, hsp) = \
            refs[ncb + 3:]
        cid = lax.axis_index("c")
        sid = lax.axis_index("s")
        wid = sid * NC + cid
        gsem = (g0, g1)
        ssem = (s0, s1)

        pltpu.sync_copy(src_hbm.at[wid], srcv)
        pltpu.sync_copy(dst_hbm.at[wid], dstv)

        def zrow(i, carry):
            for c4 in range(FB // 16):
                zbuf[i, pl.ds(c4 * 16, 16)] = jnp.zeros((16,), jnp.float32)
            return carry

        lax.fori_loop(0, C, zrow, 0)

        def fire_g(slot, grp):
            for b in range(NB):
                i = grp * NB + b
                pltpu.async_copy(hsp.at[srcv.at[i]], rows.at[slot, b],
                                 gsem[slot])

        def wait_g(slot, grp):
            for b in range(NB):
                i = grp * NB + b
                pltpu.make_async_copy(hsp.at[srcv.at[i]],
                                      rows.at[slot, b], gsem[slot]).wait()

        def run_s(slot, grp):
            ds = []
            for b in range(NB):
                i = grp * NB + b
                ds.append(pltpu.async_copy(rows.at[slot, b],
                                           acc.at[dstv.at[i]], ssem[slot],
                                           add=True))
            for d in ds:
                d.wait()

        for cb in range(ncb):
            stg = pltpu.async_copy(
                h_hbm[cb].at[pl.ds(sid * RPT, RPT)],
                hsp.at[pl.ds(sid * RPT, RPT)], hsem)
            for r in range(RPT // C):
                pltpu.sync_copy(zbuf, acc.at[pl.ds(sid * RPT + r * C, C)])
            stg.wait()
            plsc.subcore_barrier()

            fire_g(0, 0)
            fire_g(1, 1)

            def outer(j2, carry):
                j = j2 * 2
                wait_g(0, j)
                run_s(0, j)
                fire_g(0, lax.rem(j + 2, GRPS))
                wait_g(1, j + 1)
                run_s(1, j + 1)
                fire_g(1, lax.rem(j + 3, GRPS))
                return carry

            lax.fori_loop(0, GRPS // 2, outer, 0)
            wait_g(0, 0)
            wait_g(1, 1)
            plsc.subcore_barrier()
            pltpu.sync_copy(
                acc.at[pl.ds(sid * RPT, RPT)],
                out_hbm.at[pl.ds((cb * NC + cid) * NPAD + sid * RPT, RPT)],
            )
            if cb + 1 < ncb:
                plsc.subcore_barrier()

    return prop


_prop64 = _make_prop(32, 2)
_prop32 = _make_prop(32, 1)
_prop16 = _make_prop(16, 1)


@functools.partial(
    pl.kernel,
    out_type=jax.ShapeDtypeStruct((2 * NPAD,), jnp.float32),
    mesh=_MESH,
    compiler_params=_SC_PARAMS,
    scratch_types=[
        pltpu.VMEM((NCH, C), jnp.int32),
        pltpu.VMEM((C,), jnp.float32),
        pltpu.SemaphoreType.DMA,
        pltpu.VMEM_SHARED((NPAD,), jnp.float32),
    ],
)
def _deg_pass(dst_hbm, out_hbm, dstv, ones, sem, acc):
    cid = lax.axis_index("c")
    sid = lax.axis_index("s")
    wid = sid * NC + cid

    pltpu.sync_copy(dst_hbm.at[wid], dstv)
    for c4 in range(C // 16):
        ones[pl.ds(c4 * 16, 16)] = jnp.zeros((16,), jnp.float32)
    for r in range(RPT // C):
        pltpu.sync_copy(ones, acc.at[pl.ds(sid * RPT + r * C, C)])
    plsc.subcore_barrier()
    for c4 in range(C // 16):
        ones[pl.ds(c4 * 16, 16)] = jnp.ones((16,), jnp.float32)

    def body(j, carry):
        ds = []
        for b in range(8):
            i = j * 8 + b
            ds.append(pltpu.async_copy(ones, acc.at[dstv.at[i]], sem,
                                       add=True))
        for d in ds:
            d.wait()
        return carry

    lax.fori_loop(0, NCH // 8, body, 0)
    plsc.subcore_barrier()
    pltpu.sync_copy(
        acc.at[pl.ds(sid * RPT, RPT)],
        out_hbm.at[pl.ds(cid * NPAD + sid * RPT, RPT)],
    )


def _sec(s, w):
    return pl.BlockSpec((_BM, w), lambda i, s=s: (s * (NPAD // _BM) + i, 0))


def _blk(w):
    return pl.BlockSpec((_BM, w), lambda i: (i, 0))


def _full(r, c):
    return pl.BlockSpec((r, c), lambda i: (0, 0))


def _mm1s(x, W, degp):
    M, K = x.shape
    F = W.shape[1]

    def body(xr, wr, d0r, d1r, dr, hl, hr):
        dinv = lax.rsqrt(d0r[...] + d1r[...] + 1.0)
        dr[...] = dinv
        h = dinv * jnp.dot(xr[...], wr[...],
                           preferred_element_type=jnp.float32)
        hl[...] = h[:, :F // 2]
        hr[...] = h[:, F // 2:]

    return pl.pallas_call(
        body,
        grid=(M // _BM,),
        in_specs=[_blk(K), _full(K, F), _sec(0, 1), _sec(1, 1)],
        out_specs=[_blk(1), _blk(F // 2), _blk(F // 2)],
        out_shape=[
            jax.ShapeDtypeStruct((M, 1), jnp.float32),
            jax.ShapeDtypeStruct((M, F // 2), jnp.float32),
            jax.ShapeDtypeStruct((M, F // 2), jnp.float32),
        ],
    )(x, W, degp, degp)


def _layer1(a, hl, hr, dinv, b, W):
    M, FB = hl.shape
    F2 = W.shape[1]

    def body(al0r, al1r, ar0r, ar1r, hlr, hrr, dr, br, wr, o):
        tl = al0r[...] + al1r[...] + hlr[...]
        tr = ar0r[...] + ar1r[...] + hrr[...]
        t = dr[...] * jnp.concatenate([tl, tr], axis=1) + br[...]
        t = jnp.maximum(t, 0.0)
        o[...] = dr[...] * jnp.dot(t, wr[...],
                                   preferred_element_type=jnp.float32)

    return pl.pallas_call(
        body,
        grid=(M // _BM,),
        in_specs=[
            _sec(0, FB), _sec(1, FB), _sec(2, FB), _sec(3, FB),
            _blk(FB), _blk(FB), _blk(1),
            _full(1, 2 * FB), _full(2 * FB, F2),
        ],
        out_specs=_blk(F2),
        out_shape=jax.ShapeDtypeStruct((M, F2), jnp.float32),
    )(a, a, a, a, hl, hr, dinv, b, W)


def _layer(a, h, dinv, b, W):
    M, F = h.shape
    F2 = W.shape[1]

    def body(a0r, a1r, hr, dr, br, wr, o):
        t = dr[...] * (a0r[...] + a1r[...] + hr[...]) + br[...]
        t = jnp.maximum(t, 0.0)
        o[...] = dr[...] * jnp.dot(t, wr[...],
                                   preferred_element_type=jnp.float32)

    return pl.pallas_call(
        body,
        grid=(M // _BM,),
        in_specs=[
            _sec(0, F), _sec(1, F), _blk(F), _blk(1),
            _full(1, F), _full(F, F2),
        ],
        out_specs=_blk(F2),
        out_shape=jax.ShapeDtypeStruct((M, F2), jnp.float32),
    )(a, a, h, dinv, b, W)


def _final(a, h, dinv, b):
    M, F = h.shape

    def body(a0r, a1r, hr, dr, br, o):
        t = dr[...] * (a0r[...] + a1r[...] + hr[...]) + br[...]
        o[...] = jax.nn.sigmoid(t)

    return pl.pallas_call(
        body,
        grid=(M // _BM,),
        in_specs=[
            _sec(0, F), _sec(1, F), _blk(F), _blk(1), _full(1, F),
        ],
        out_specs=_blk(F),
        out_shape=jax.ShapeDtypeStruct((M, F), jnp.float32),
    )(a, a, h, dinv, b)


def kernel(x, edge_index, W1, b1, W2, b2, W3, b3):
    ei = edge_index.astype(jnp.int32)
    pad = EPAD - E
    src = jnp.concatenate([ei[0], jnp.zeros((pad,), jnp.int32)])
    dst = jnp.concatenate(
        [ei[1], N + (jnp.arange(pad, dtype=jnp.int32) % (NPAD - N))])
    src = src.reshape(NW, NCH, C)
    dst = dst.reshape(NW, NCH, C)
    xp = jnp.pad(x, ((0, NPAD - N), (0, 0)))

    degp = _deg_pass(dst).reshape(2 * NPAD, 1)
    dinv, h1l, h1r = _mm1s(xp, W1, degp)

    a = _prop64(h1l, h1r, src, dst)
    h2 = _layer1(a, h1l, h1r, dinv, b1.reshape(1, -1), W2)

    a = _prop32(h2, src, dst)
    h3 = _layer(a, h2, dinv, b2.reshape(1, -1), W3)

    a = _prop16(h3, src, dst)
    return _final(a, h3, dinv, b3.reshape(1, -1))[:N]


# speedup vs baseline: 45.3782x; 1.1586x over previous
import functools

import jax
import jax.numpy as jnp
from jax import lax
from jax.experimental import pallas as pl
from jax.experimental.pallas import tpu as pltpu
from jax.experimental.pallas import tpu_sc as plsc

N = 10000
E = 320000
NC, NS = 2, 16
NW = NC * NS
C = 128
NCH = 80
EPT = NCH * C
EPAD = NW * EPT
NPAD = 10240
RPT = NPAD // NS
NB = 4
GRPS = NCH // NB
_BM = 1024

_MESH = plsc.VectorSubcoreMesh(core_axis_name="c", subcore_axis_name="s")
_SC_PARAMS = pltpu.CompilerParams(use_tc_tiling_on_sc=False)


def _make_prop(FB, ncb):
    @functools.partial(
        pl.kernel,
        out_type=jax.ShapeDtypeStruct((NPAD, 128), jnp.float32),
        mesh=_MESH,
        compiler_params=_SC_PARAMS,
        scratch_types=[
            pltpu.VMEM((NCH, C), jnp.int32),
            pltpu.VMEM((NCH, C), jnp.int32),
            pltpu.VMEM((2, NB, C, FB), jnp.float32),
            pltpu.VMEM((C, FB), jnp.float32),
            pltpu.SemaphoreType.DMA,
            pltpu.SemaphoreType.DMA,
            pltpu.SemaphoreType.DMA,
            pltpu.SemaphoreType.DMA,
            pltpu.SemaphoreType.DMA,
            pltpu.VMEM_SHARED((NPAD, FB), jnp.float32),
            pltpu.VMEM_SHARED((NPAD, FB), jnp.float32),
        ],
    )
    def prop(h_hbm, src_hbm, dst_hbm, out_hbm, srcv, dstv, rows, zbuf,
             g0, g1, s0, s1, hsem, acc, hsp):
        cid = lax.axis_index("c")
        sid = lax.axis_index("s")
        wid = sid * NC + cid
        gsem = (g0, g1)
        ssem = (s0, s1)

        pltpu.sync_copy(src_hbm.at[wid], srcv)
        pltpu.sync_copy(dst_hbm.at[wid], dstv)

        def zrow(i, carry):
            for c4 in range(FB // 16):
                zbuf[i, pl.ds(c4 * 16, 16)] = jnp.zeros((16,), jnp.float32)
            return carry

        lax.fori_loop(0, C, zrow, 0)

        def fire_g(slot, grp):
            for b in range(NB):
                i = grp * NB + b
                pltpu.async_copy(hsp.at[srcv.at[i]], rows.at[slot, b],
                                 gsem[slot])

        def wait_g(slot, grp):
            for b in range(NB):
                i = grp * NB + b
                pltpu.make_async_copy(hsp.at[srcv.at[i]],
                                      rows.at[slot, b], gsem[slot]).wait()

        def run_s(slot, grp):
            ds = []
            for b in range(NB):
                i = grp * NB + b
                ds.append(pltpu.async_copy(rows.at[slot, b],
                                           acc.at[dstv.at[i]], ssem[slot],
                                           add=True))
            for d in ds:
                d.wait()

        for cb in range(ncb):
            stg = pltpu.async_copy(
                h_hbm.at[pl.ds(sid * RPT, RPT), pl.ds(cb * FB, FB)],
                hsp.at[pl.ds(sid * RPT, RPT)], hsem)
            for r in range(RPT // C):
                pltpu.sync_copy(zbuf, acc.at[pl.ds(sid * RPT + r * C, C)])
            stg.wait()
            plsc.subcore_barrier()

            fire_g(0, 0)
            fire_g(1, 1)

            def outer(j2, carry):
                j = j2 * 2
                wait_g(0, j)
                run_s(0, j)
                fire_g(0, lax.rem(j + 2, GRPS))
                wait_g(1, j + 1)
                run_s(1, j + 1)
                fire_g(1, lax.rem(j + 3, GRPS))
                return carry

            lax.fori_loop(0, GRPS // 2, outer, 0)
            wait_g(0, 0)
            wait_g(1, 1)
            plsc.subcore_barrier()
            pltpu.sync_copy(
                acc.at[pl.ds(sid * RPT, RPT)],
                out_hbm.at[pl.ds(sid * RPT, RPT),
                           pl.ds((cb * NC + cid) * FB, FB)],
            )
            if cb + 1 < ncb:
                plsc.subcore_barrier()

    return prop


_prop64 = _make_prop(32, 2)
_prop32 = _make_prop(32, 1)
_prop16 = _make_prop(16, 1)


@functools.partial(
    pl.kernel,
    out_type=jax.ShapeDtypeStruct((2 * NPAD,), jnp.float32),
    mesh=_MESH,
    compiler_params=_SC_PARAMS,
    scratch_types=[
        pltpu.VMEM((NCH, C), jnp.int32),
        pltpu.VMEM((C,), jnp.float32),
        pltpu.SemaphoreType.DMA,
        pltpu.VMEM_SHARED((NPAD,), jnp.float32),
    ],
)
def _deg_pass(dst_hbm, out_hbm, dstv, ones, sem, acc):
    cid = lax.axis_index("c")
    sid = lax.axis_index("s")
    wid = sid * NC + cid

    pltpu.sync_copy(dst_hbm.at[wid], dstv)
    for c4 in range(C // 16):
        ones[pl.ds(c4 * 16, 16)] = jnp.zeros((16,), jnp.float32)
    for r in range(RPT // C):
        pltpu.sync_copy(ones, acc.at[pl.ds(sid * RPT + r * C, C)])
    plsc.subcore_barrier()
    for c4 in range(C // 16):
        ones[pl.ds(c4 * 16, 16)] = jnp.ones((16,), jnp.float32)

    def body(j, carry):
        ds = []
        for b in range(8):
            i = j * 8 + b
            ds.append(pltpu.async_copy(ones, acc.at[dstv.at[i]], sem,
                                       add=True))
        for d in ds:
            d.wait()
        return carry

    lax.fori_loop(0, NCH // 8, body, 0)
    plsc.subcore_barrier()
    pltpu.sync_copy(
        acc.at[pl.ds(sid * RPT, RPT)],
        out_hbm.at[pl.ds(cid * NPAD + sid * RPT, RPT)],
    )


def _blk(w, bm=_BM):
    return pl.BlockSpec((bm, w), lambda i: (i, 0))


def _full(r, c):
    return pl.BlockSpec((r, c), lambda i: (0, 0))


def _mm1s(x, W, degp):
    M, K = x.shape
    F = W.shape[1]

    def body(xr, wr, d0r, d1r, o):
        dinv = lax.rsqrt(d0r[...] + d1r[...] + 1.0)
        h = dinv * jnp.dot(xr[...], wr[...],
                           preferred_element_type=jnp.float32)
        o[...] = jnp.concatenate(
            [h, dinv, jnp.zeros((h.shape[0], 128 - F - 1), jnp.float32)],
            axis=1)

    nsec = NPAD // _BM
    return pl.pallas_call(
        body,
        grid=(M // _BM,),
        in_specs=[
            _blk(K), _full(K, F),
            pl.BlockSpec((_BM, 1), lambda i: (i, 0)),
            pl.BlockSpec((_BM, 1), lambda i: (nsec + i, 0)),
        ],
        out_specs=_blk(128),
        out_shape=jax.ShapeDtypeStruct((M, 128), jnp.float32),
    )(x, W, degp, degp)


def _layer1(a, p1, b, W):
    F2 = W.shape[1]

    def body(ar, pr, br, wr, o):
        av = ar[...]
        pv = pr[...]
        dinv = pv[:, 64:65]
        acc = jnp.concatenate(
            [av[:, 0:32] + av[:, 32:64], av[:, 64:96] + av[:, 96:128]],
            axis=1)
        t = dinv * (acc + pv[:, 0:64]) + br[...]
        t = jnp.maximum(t, 0.0)
        h = dinv * jnp.dot(t, wr[...], preferred_element_type=jnp.float32)
        o[...] = jnp.concatenate(
            [h, jnp.zeros((h.shape[0], 128 - F2), jnp.float32)], axis=1)

    return pl.pallas_call(
        body,
        grid=(NPAD // _BM,),
        in_specs=[_blk(128), _blk(128), _full(1, 64), _full(64, F2)],
        out_specs=_blk(128),
        out_shape=jax.ShapeDtypeStruct((NPAD, 128), jnp.float32),
    )(a, p1, b, W)


def _layer2(a, p2, p1, b, W):
    F = 32
    F2 = W.shape[1]

    def body(ar, p2r, p1r, br, wr, o):
        av = ar[...]
        dinv = p1r[...][:, 64:65]
        t = dinv * (av[:, 0:F] + av[:, F:2 * F] + p2r[...][:, 0:F]) + br[...]
        t = jnp.maximum(t, 0.0)
        h = dinv * jnp.dot(t, wr[...], preferred_element_type=jnp.float32)
        o[...] = jnp.concatenate(
            [h, jnp.zeros((h.shape[0], 128 - F2), jnp.float32)], axis=1)

    return pl.pallas_call(
        body,
        grid=(NPAD // _BM,),
        in_specs=[
            _blk(128), _blk(128), _blk(128),
            _full(1, F), _full(F, F2),
        ],
        out_specs=_blk(128),
        out_shape=jax.ShapeDtypeStruct((NPAD, 128), jnp.float32),
    )(a, p2, p1, b, W)


def _final(a, p3, p1, b):
    F = 16
    bm = 1000

    def body(ar, p3r, p1r, br, o):
        av = ar[...]
        dinv = p1r[...][:, 64:65]
        t = dinv * (av[:, 0:F] + av[:, F:2 * F] + p3r[...][:, 0:F]) + br[...]
        o[...] = jax.nn.sigmoid(t)

    return pl.pallas_call(
        body,
        grid=(N // bm,),
        in_specs=[
            _blk(128, bm), _blk(128, bm), _blk(128, bm),
            _full(1, F),
        ],
        out_specs=_blk(F, bm),
        out_shape=jax.ShapeDtypeStruct((N, F), jnp.float32),
    )(a, p3, p1, b)


def kernel(x, edge_index, W1, b1, W2, b2, W3, b3):
    ei = edge_index.astype(jnp.int32)
    pad = EPAD - E
    src = jnp.concatenate([ei[0], jnp.zeros((pad,), jnp.int32)])
    dst = jnp.concatenate(
        [ei[1], N + (jnp.arange(pad, dtype=jnp.int32) % (NPAD - N))])
    src = src.reshape(NW, NCH, C)
    dst = dst.reshape(NW, NCH, C)
    xp = jnp.pad(x, ((0, NPAD - N), (0, 0)))

    degp = _deg_pass(dst).reshape(2 * NPAD, 1)
    p1 = _mm1s(xp, W1, degp)

    a = _prop64(p1, src, dst)
    p2 = _layer1(a, p1, b1.reshape(1, -1), W2)

    a = _prop32(p2, src, dst)
    p3 = _layer2(a, p2, p1, b2.reshape(1, -1), W3)

    a = _prop16(p3, src, dst)
    return _final(a, p3, p1, b3.reshape(1, -1))


# speedup vs baseline: 46.4906x; 1.0245x over previous
import functools

import jax
import jax.numpy as jnp
from jax import lax
from jax.experimental import pallas as pl
from jax.experimental.pallas import tpu as pltpu
from jax.experimental.pallas import tpu_sc as plsc

N = 10000
E = 320000
NC, NS = 2, 16
NW = NC * NS
C = 128
NCH = 80
EPT = NCH * C
EPAD = NW * EPT
NPAD = 10240
RPT = NPAD // NS
NB = 4
GRPS = NCH // NB
_BM = 2048

_MESH = plsc.VectorSubcoreMesh(core_axis_name="c", subcore_axis_name="s")
_SC_PARAMS = pltpu.CompilerParams(use_tc_tiling_on_sc=False)


def _make_prop(FB, ncb):
    @functools.partial(
        pl.kernel,
        out_type=jax.ShapeDtypeStruct((NPAD, 128), jnp.float32),
        mesh=_MESH,
        compiler_params=_SC_PARAMS,
        scratch_types=[
            pltpu.VMEM((NCH, C), jnp.int32),
            pltpu.VMEM((NCH, C), jnp.int32),
            pltpu.VMEM((2, NB, C, FB), jnp.float32),
            pltpu.VMEM((C, FB), jnp.float32),
            pltpu.SemaphoreType.DMA,
            pltpu.SemaphoreType.DMA,
            pltpu.SemaphoreType.DMA,
            pltpu.SemaphoreType.DMA,
            pltpu.SemaphoreType.DMA,
            pltpu.VMEM_SHARED((NPAD, FB), jnp.float32),
            pltpu.VMEM_SHARED((NPAD, FB), jnp.float32),
        ],
    )
    def prop(h_hbm, src_hbm, dst_hbm, out_hbm, srcv, dstv, rows, zbuf,
             g0, g1, s0, s1, hsem, acc, hsp):
        cid = lax.axis_index("c")
        sid = lax.axis_index("s")
        wid = sid * NC + cid
        gsem = (g0, g1)
        ssem = (s0, s1)

        pltpu.sync_copy(src_hbm.at[wid], srcv)
        pltpu.sync_copy(dst_hbm.at[wid], dstv)

        def zrow(i, carry):
            for c4 in range(FB // 16):
                zbuf[i, pl.ds(c4 * 16, 16)] = jnp.zeros((16,), jnp.float32)
            return carry

        lax.fori_loop(0, C, zrow, 0)

        def fire_g(slot, grp):
            for b in range(NB):
                i = grp * NB + b
                pltpu.async_copy(hsp.at[srcv.at[i]], rows.at[slot, b],
                                 gsem[slot])

        def wait_g(slot, grp):
            for b in range(NB):
                i = grp * NB + b
                pltpu.make_async_copy(hsp.at[srcv.at[i]],
                                      rows.at[slot, b], gsem[slot]).wait()

        def run_s(slot, grp):
            ds = []
            for b in range(NB):
                i = grp * NB + b
                ds.append(pltpu.async_copy(rows.at[slot, b],
                                           acc.at[dstv.at[i]], ssem[slot],
                                           add=True))
            for d in ds:
                d.wait()

        for cb in range(ncb):
            stg = pltpu.async_copy(
                h_hbm.at[pl.ds(sid * RPT, RPT), pl.ds(cb * FB, FB)],
                hsp.at[pl.ds(sid * RPT, RPT)], hsem)
            for r in range(RPT // C):
                pltpu.sync_copy(zbuf, acc.at[pl.ds(sid * RPT + r * C, C)])
            stg.wait()
            plsc.subcore_barrier()

            fire_g(0, 0)
            fire_g(1, 1)

            def outer(j2, carry):
                j = j2 * 2
                wait_g(0, j)
                run_s(0, j)
                fire_g(0, lax.rem(j + 2, GRPS))
                wait_g(1, j + 1)
                run_s(1, j + 1)
                fire_g(1, lax.rem(j + 3, GRPS))
                return carry

            lax.fori_loop(0, GRPS // 2, outer, 0)
            wait_g(0, 0)
            wait_g(1, 1)
            plsc.subcore_barrier()
            pltpu.sync_copy(
                acc.at[pl.ds(sid * RPT, RPT)],
                out_hbm.at[pl.ds(sid * RPT, RPT),
                           pl.ds((cb * NC + cid) * FB, FB)],
            )
            if cb + 1 < ncb:
                plsc.subcore_barrier()

    return prop


_prop64 = _make_prop(32, 2)
_prop32 = _make_prop(32, 1)
_prop16 = _make_prop(16, 1)


@functools.partial(
    pl.kernel,
    out_type=jax.ShapeDtypeStruct((2 * NPAD,), jnp.float32),
    mesh=_MESH,
    compiler_params=_SC_PARAMS,
    scratch_types=[
        pltpu.VMEM((NCH, C), jnp.int32),
        pltpu.VMEM((C,), jnp.float32),
        pltpu.SemaphoreType.DMA,
        pltpu.VMEM_SHARED((NPAD,), jnp.float32),
    ],
)
def _deg_pass(dst_hbm, out_hbm, dstv, ones, sem, acc):
    cid = lax.axis_index("c")
    sid = lax.axis_index("s")
    wid = sid * NC + cid

    pltpu.sync_copy(dst_hbm.at[wid], dstv)
    for c4 in range(C // 16):
        ones[pl.ds(c4 * 16, 16)] = jnp.zeros((16,), jnp.float32)
    for r in range(RPT // C):
        pltpu.sync_copy(ones, acc.at[pl.ds(sid * RPT + r * C, C)])
    plsc.subcore_barrier()
    for c4 in range(C // 16):
        ones[pl.ds(c4 * 16, 16)] = jnp.ones((16,), jnp.float32)

    def body(j, carry):
        ds = []
        for b in range(8):
            i = j * 8 + b
            ds.append(pltpu.async_copy(ones, acc.at[dstv.at[i]], sem,
                                       add=True))
        for d in ds:
            d.wait()
        return carry

    lax.fori_loop(0, NCH // 8, body, 0)
    plsc.subcore_barrier()
    pltpu.sync_copy(
        acc.at[pl.ds(sid * RPT, RPT)],
        out_hbm.at[pl.ds(cid * NPAD + sid * RPT, RPT)],
    )


def _blk(w, bm=_BM):
    return pl.BlockSpec((bm, w), lambda i: (i, 0))


def _full(r, c):
    return pl.BlockSpec((r, c), lambda i: (0, 0))


def _mm1s(x, W, degp):
    M, K = x.shape
    F = W.shape[1]

    def body(xr, wr, d0r, d1r, o):
        dinv = lax.rsqrt(d0r[...] + d1r[...] + 1.0)
        h = dinv * jnp.dot(xr[...], wr[...],
                           preferred_element_type=jnp.float32)
        o[...] = jnp.concatenate(
            [h, dinv, jnp.zeros((h.shape[0], 128 - F - 1), jnp.float32)],
            axis=1)

    nsec = NPAD // _BM
    return pl.pallas_call(
        body,
        grid=(M // _BM,),
        in_specs=[
            _blk(K), _full(K, F),
            pl.BlockSpec((_BM, 1), lambda i: (i, 0)),
            pl.BlockSpec((_BM, 1), lambda i: (nsec + i, 0)),
        ],
        out_specs=_blk(128),
        out_shape=jax.ShapeDtypeStruct((M, 128), jnp.float32),
    )(x, W, degp, degp)


def _layer1(a, p1, b, W):
    F2 = W.shape[1]

    def body(ar, pr, br, wr, o):
        av = ar[...]
        pv = pr[...]
        dinv = pv[:, 64:65]
        acc = jnp.concatenate(
            [av[:, 0:32] + av[:, 32:64], av[:, 64:96] + av[:, 96:128]],
            axis=1)
        t = dinv * (acc + pv[:, 0:64]) + br[...]
        t = jnp.maximum(t, 0.0)
        h = dinv * jnp.dot(t, wr[...], preferred_element_type=jnp.float32)
        o[...] = jnp.concatenate(
            [h, jnp.zeros((h.shape[0], 128 - F2), jnp.float32)], axis=1)

    return pl.pallas_call(
        body,
        grid=(NPAD // _BM,),
        in_specs=[_blk(128), _blk(128), _full(1, 64), _full(64, F2)],
        out_specs=_blk(128),
        out_shape=jax.ShapeDtypeStruct((NPAD, 128), jnp.float32),
    )(a, p1, b, W)


def _layer2(a, p2, p1, b, W):
    F = 32
    F2 = W.shape[1]

    def body(ar, p2r, p1r, br, wr, o):
        av = ar[...]
        dinv = p1r[...][:, 64:65]
        t = dinv * (av[:, 0:F] + av[:, F:2 * F] + p2r[...][:, 0:F]) + br[...]
        t = jnp.maximum(t, 0.0)
        h = dinv * jnp.dot(t, wr[...], preferred_element_type=jnp.float32)
        o[...] = jnp.concatenate(
            [h, jnp.zeros((h.shape[0], 128 - F2), jnp.float32)], axis=1)

    return pl.pallas_call(
        body,
        grid=(NPAD // _BM,),
        in_specs=[
            _blk(128), _blk(128), _blk(128),
            _full(1, F), _full(F, F2),
        ],
        out_specs=_blk(128),
        out_shape=jax.ShapeDtypeStruct((NPAD, 128), jnp.float32),
    )(a, p2, p1, b, W)


def _final(a, p3, p1, b):
    F = 16
    bm = 1000

    def body(ar, p3r, p1r, br, o):
        av = ar[...]
        dinv = p1r[...][:, 64:65]
        t = dinv * (av[:, 0:F] + av[:, F:2 * F] + p3r[...][:, 0:F]) + br[...]
        o[...] = jax.nn.sigmoid(t)

    return pl.pallas_call(
        body,
        grid=(N // bm,),
        in_specs=[
            _blk(128, bm), _blk(128, bm), _blk(128, bm),
            _full(1, F),
        ],
        out_specs=_blk(F, bm),
        out_shape=jax.ShapeDtypeStruct((N, F), jnp.float32),
    )(a, p3, p1, b)


def kernel(x, edge_index, W1, b1, W2, b2, W3, b3):
    ei = edge_index.astype(jnp.int32)
    pad = EPAD - E
    src = jnp.concatenate([ei[0], jnp.zeros((pad,), jnp.int32)])
    dst = jnp.concatenate(
        [ei[1], N + (jnp.arange(pad, dtype=jnp.int32) % (NPAD - N))])
    src = src.reshape(NW, NCH, C)
    dst = dst.reshape(NW, NCH, C)
    xp = jnp.pad(x, ((0, NPAD - N), (0, 0)))

    degp = _deg_pass(dst).reshape(2 * NPAD, 1)
    p1 = _mm1s(xp, W1, degp)

    a = _prop64(p1, src, dst)
    p2 = _layer1(a, p1, b1.reshape(1, -1), W2)

    a = _prop32(p2, src, dst)
    p3 = _layer2(a, p2, p1, b2.reshape(1, -1), W3)

    a = _prop16(p3, src, dst)
    return _final(a, p3, p1, b3.reshape(1, -1))
